# baseline (speedup 1.0000x reference)
import functools

import jax
import jax.numpy as jnp
from jax import lax
from jax.experimental import pallas as pl
from jax.experimental.pallas import tpu as pltpu
from jax.experimental.pallas import tpu_sc as plsc

N_NODES = 10000
NP = 10016
N_EDGES = 160000
IN_CH = 128
HID = 16
KH = 64
PW = HID * KH
MSGW = 128

CHUNK = 128
NW = 32
CPW = 40
EP = NW * CPW * CHUNK
NCHUNKS = EP // CHUNK

BE = 1024
BN = 1000


@functools.lru_cache(maxsize=None)
def _sc_kernels():
    mesh = plsc.VectorSubcoreMesh(core_axis_name="c", subcore_axis_name="s")

    @functools.partial(
        pl.kernel,
        out_type=jax.ShapeDtypeStruct((EP, IN_CH), jnp.float32),
        mesh=mesh,
        scratch_types=[
            pltpu.VMEM((CPW, CHUNK), jnp.int32),
            pltpu.VMEM((2, CHUNK, IN_CH), jnp.float32),
            pltpu.SemaphoreType.DMA,
            pltpu.SemaphoreType.DMA,
            pltpu.SemaphoreType.DMA,
            pltpu.SemaphoreType.DMA,
        ],
    )
    def _sc_gather(x_hbm, src2d_hbm, out_hbm, idx_all, rows_v, g0, g1, o0, o1):
        wid = lax.axis_index("s") * 2 + lax.axis_index("c")
        first = wid * CPW
        pltpu.sync_copy(src2d_hbm.at[pl.ds(first, CPW)], idx_all)

        def start_gather(i, g, b):
            pltpu.async_copy(x_hbm.at[idx_all.at[i]], rows_v.at[b], g)

        def wait_gather(i, g, b):
            pltpu.make_async_copy(x_hbm.at[idx_all.at[i]], rows_v.at[b],
                                  g).wait()

        def start_out(i, o, b):
            pltpu.async_copy(rows_v.at[b],
                             out_hbm.at[pl.ds((first + i) * CHUNK, CHUNK)], o)

        def wait_out(i, o, b):
            pltpu.make_async_copy(rows_v.at[b],
                                  out_hbm.at[pl.ds((first + i) * CHUNK, CHUNK)],
                                  o).wait()

        start_gather(0, g0, 0)

        def body(it, carry):
            i0 = 2 * it
            i1 = i0 + 1

            @pl.when(it >= 1)
            def _():
                wait_out(i0 - 1, o1, 1)

            start_gather(i1, g1, 1)
            wait_gather(i0, g0, 0)
            start_out(i0, o0, 0)

            @pl.when(i1 + 1 < CPW)
            def _():
                wait_out(i0, o0, 0)
                start_gather(i1 + 1, g0, 0)

            wait_gather(i1, g1, 1)
            start_out(i1, o1, 1)
            return carry

        lax.fori_loop(0, CPW // 2, body, 0)
        wait_out(CPW - 1, o1, 1)
        wait_out(CPW - 2, o0, 0)

    @functools.partial(
        pl.kernel,
        out_type=jax.ShapeDtypeStruct((2, NP, MSGW), jnp.float32),
        mesh=mesh,
        scratch_types=[
            pltpu.VMEM((CPW, CHUNK), jnp.int32),
            pltpu.VMEM((2, CHUNK, MSGW), jnp.float32),
            pltpu.SemaphoreType.DMA,
            pltpu.SemaphoreType.DMA,
            pltpu.VMEM_SHARED((NP, MSGW), jnp.float32),
        ],
    )
    def _sc_scatter(msg_hbm, dst2d_hbm, zeros_hbm, out_hbm, idx_all, buf_v,
                    l0, l1, acc_sh):
        cid = lax.axis_index("c")
        sid = lax.axis_index("s")
        wid = sid * 2 + cid
        first = wid * CPW

        pltpu.sync_copy(dst2d_hbm.at[pl.ds(first, CPW)], idx_all)

        @pl.when(sid == 0)
        def _():
            pltpu.sync_copy(zeros_hbm, acc_sh)

        plsc.subcore_barrier()

        def start_load(i, l, b):
            pltpu.async_copy(msg_hbm.at[pl.ds((first + i) * CHUNK, CHUNK)],
                             buf_v.at[b], l)

        def wait_load(i, l, b):
            pltpu.make_async_copy(msg_hbm.at[pl.ds((first + i) * CHUNK, CHUNK)],
                                  buf_v.at[b], l).wait()

        def scatter(i, b):
            pltpu.sync_copy(buf_v.at[b], acc_sh.at[idx_all.at[i]], add=True)

        start_load(0, l0, 0)

        def body(it, carry):
            i0 = 2 * it
            i1 = i0 + 1
            start_load(i1, l1, 1)
            wait_load(i0, l0, 0)
            scatter(i0, 0)

            @pl.when(i1 + 1 < CPW)
            def _():
                start_load(i1 + 1, l0, 0)

            wait_load(i1, l1, 1)
            scatter(i1, 1)
            return carry

        lax.fori_loop(0, CPW // 2, body, 0)
        plsc.subcore_barrier()

        @pl.when(sid == 0)
        def _():
            pltpu.sync_copy(acc_sh, out_hbm.at[cid])

    return _sc_gather, _sc_scatter


def _edge_body(ea_ref, xj_ref, w1_ref, b1_ref, t_ref, s_ref, out_ref):
    h = jnp.dot(ea_ref[...], w1_ref[...], preferred_element_type=jnp.float32)
    h = jnp.maximum(h + b1_ref[...], 0.0)
    p = jnp.dot(xj_ref[...], t_ref[...], preferred_element_type=jnp.float32)
    htile = jnp.concatenate([h] * HID, axis=1)
    g = p[:, :PW] * htile
    msg = jnp.dot(g, s_ref[...], preferred_element_type=jnp.float32)
    msg = msg + p[:, PW:]
    onec = (lax.broadcasted_iota(jnp.int32, (BE, HID), 1) == 0)
    pad = jnp.zeros((BE, MSGW - 2 * HID), dtype=jnp.float32)
    out_ref[...] = jnp.concatenate([msg, onec.astype(jnp.float32), pad], axis=1)


def _edge_kernel(ea_pad, xj, w1, b1_2d, t2, s):
    return pl.pallas_call(
        _edge_body,
        grid=(EP // BE,),
        in_specs=[
            pl.BlockSpec((BE, HID), lambda i: (i, 0)),
            pl.BlockSpec((BE, IN_CH), lambda i: (i, 0)),
            pl.BlockSpec((HID, KH), lambda i: (0, 0)),
            pl.BlockSpec((1, KH), lambda i: (0, 0)),
            pl.BlockSpec((IN_CH, PW + HID), lambda i: (0, 0)),
            pl.BlockSpec((PW, HID), lambda i: (0, 0)),
        ],
        out_specs=pl.BlockSpec((BE, MSGW), lambda i: (i, 0)),
        out_shape=jax.ShapeDtypeStruct((EP, MSGW), jnp.float32),
    )(ea_pad, xj, w1, b1_2d, t2, s)


def _final_body(p_ref, x_ref, root_ref, bias_ref, wfc_ref, bfc_ref, out_ref):
    s = p_ref[0] + p_ref[1]
    sums = s[:, :HID]
    cnt = s[:, HID:HID + 1]
    aggr = sums / jnp.maximum(cnt, 1.0)
    conv = aggr + jnp.dot(x_ref[...], root_ref[...],
                          preferred_element_type=jnp.float32) + bias_ref[...]
    conv = jnp.maximum(conv, 0.0)
    out_ref[...] = jnp.dot(conv, wfc_ref[...],
                           preferred_element_type=jnp.float32) + bfc_ref[...]


def _final_kernel(partials, x, root, bias_2d, wfc, bfc_2d):
    return pl.pallas_call(
        _final_body,
        grid=(N_NODES // BN,),
        in_specs=[
            pl.BlockSpec((2, BN, MSGW), lambda i: (0, i, 0)),
            pl.BlockSpec((BN, IN_CH), lambda i: (i, 0)),
            pl.BlockSpec((IN_CH, HID), lambda i: (0, 0)),
            pl.BlockSpec((1, HID), lambda i: (0, 0)),
            pl.BlockSpec((HID, 1), lambda i: (0, 0)),
            pl.BlockSpec((1, 1), lambda i: (0, 0)),
        ],
        out_specs=pl.BlockSpec((BN, 1), lambda i: (i, 0)),
        out_shape=jax.ShapeDtypeStruct((N_NODES, 1), jnp.float32),
    )(partials, x, root, bias_2d, wfc, bfc_2d)


def kernel(x, edge_index, edge_attr, W1, b1, W2, b2, root, bias, Wfc, bfc):
    src = edge_index[0].astype(jnp.int32)
    dst = edge_index[1].astype(jnp.int32)
    npad = EP - N_EDGES
    src_pad = jnp.concatenate([src, jnp.zeros((npad,), jnp.int32)])
    dst_pad = jnp.concatenate([dst, jnp.full((npad,), N_NODES, jnp.int32)])
    ea_pad = jnp.concatenate(
        [edge_attr, jnp.zeros((npad, HID), jnp.float32)])

    t = jnp.transpose(W2.reshape(KH, IN_CH, HID), (1, 2, 0)).reshape(IN_CH, PW)
    t2 = jnp.concatenate([t, b2.reshape(IN_CH, HID)], axis=1)
    s = jnp.repeat(jnp.eye(HID, dtype=jnp.float32), KH, axis=0)

    sc_gather, sc_scatter = _sc_kernels()
    xj = sc_gather(x, src_pad.reshape(NCHUNKS, CHUNK))
    msg = _edge_kernel(ea_pad, xj, W1, b1.reshape(1, KH), t2, s)
    zeros = jnp.zeros((NP, MSGW), dtype=jnp.float32)
    partials = sc_scatter(msg, dst_pad.reshape(NCHUNKS, CHUNK), zeros)
    return _final_kernel(partials, x, root, bias.reshape(1, HID),
                         Wfc, bfc.reshape(1, 1))


# speedup vs baseline: 1.3900x; 1.3900x over previous
import functools

import jax
import jax.numpy as jnp
from jax import lax
from jax.experimental import pallas as pl
from jax.experimental.pallas import tpu as pltpu
from jax.experimental.pallas import tpu_sc as plsc

N_NODES = 10000
N_EDGES = 160000
IN_CH = 128
HID = 16
KH = 64
PW = HID * KH
MSGW = 128

CHUNK = 128
NW = 32
NCHUNKS = N_EDGES // CHUNK
CPW = (NCHUNKS + NW - 1) // NW

BE = 1000
BN = 1000


@functools.lru_cache(maxsize=None)
def _sc_kernels():
    mesh = plsc.VectorSubcoreMesh(core_axis_name="c", subcore_axis_name="s")

    @functools.partial(
        pl.kernel,
        out_type=jax.ShapeDtypeStruct((N_EDGES, IN_CH), jnp.float32),
        mesh=mesh,
        scratch_types=[
            pltpu.VMEM((CHUNK,), jnp.int32),
            pltpu.VMEM((CHUNK, IN_CH), jnp.float32),
            pltpu.SemaphoreType.DMA,
        ],
    )
    def _sc_gather(x_hbm, src_hbm, out_hbm, idx_v, rows_v, sem):
        wid = lax.axis_index("s") * 2 + lax.axis_index("c")

        def body(i, carry):
            j = wid + i * NW

            @pl.when(j < NCHUNKS)
            def _():
                base = j * CHUNK
                pltpu.sync_copy(src_hbm.at[pl.ds(base, CHUNK)], idx_v)
                pltpu.async_copy(x_hbm.at[idx_v], rows_v, sem).wait()
                pltpu.sync_copy(rows_v, out_hbm.at[pl.ds(base, CHUNK)])

            return carry

        lax.fori_loop(0, CPW, body, 0)

    @functools.partial(
        pl.kernel,
        out_type=jax.ShapeDtypeStruct((2, N_NODES, MSGW), jnp.float32),
        mesh=mesh,
        scratch_types=[
            pltpu.VMEM((CHUNK,), jnp.int32),
            pltpu.VMEM((CHUNK,), jnp.int32),
            pltpu.VMEM((CHUNK, MSGW), jnp.float32),
            pltpu.VMEM((CHUNK, MSGW), jnp.float32),
            pltpu.SemaphoreType.DMA,
            pltpu.SemaphoreType.DMA,
            pltpu.SemaphoreType.DMA,
            pltpu.SemaphoreType.DMA,
            pltpu.VMEM_SHARED((N_NODES, MSGW), jnp.float32),
        ],
    )
    def _sc_scatter(msg_hbm, dst_hbm, zeros_hbm, out_hbm, idx_a, idx_b,
                    buf_a, buf_b, la, lb, ia, ib, acc_sh):
        cid = lax.axis_index("c")
        sid = lax.axis_index("s")
        wid = sid * 2 + cid

        @pl.when(sid == 0)
        def _():
            pltpu.sync_copy(zeros_hbm, acc_sh)

        plsc.subcore_barrier()

        def chunk(i):
            return wid + i * NW

        def start_load(i, buf, idx, l, isem):
            j = chunk(i)
            pltpu.async_copy(msg_hbm.at[pl.ds(j * CHUNK, CHUNK)], buf, l)
            pltpu.async_copy(dst_hbm.at[pl.ds(j * CHUNK, CHUNK)], idx, isem)

        def wait_load(i, buf, idx, l, isem):
            j = chunk(i)
            pltpu.make_async_copy(msg_hbm.at[pl.ds(j * CHUNK, CHUNK)], buf,
                                  l).wait()
            pltpu.make_async_copy(dst_hbm.at[pl.ds(j * CHUNK, CHUNK)], idx,
                                  isem).wait()

        def scatter(buf, idx):
            pltpu.sync_copy(buf, acc_sh.at[idx], add=True)

        @pl.when(chunk(0) < NCHUNKS)
        def _():
            start_load(0, buf_a, idx_a, la, ia)

        def body(it, carry):
            i0 = 2 * it
            i1 = i0 + 1

            @pl.when(chunk(i1) < NCHUNKS)
            def _():
                start_load(i1, buf_b, idx_b, lb, ib)

            @pl.when(chunk(i0) < NCHUNKS)
            def _():
                wait_load(i0, buf_a, idx_a, la, ia)
                scatter(buf_a, idx_a)

            @pl.when(chunk(i1 + 1) < NCHUNKS)
            def _():
                start_load(i1 + 1, buf_a, idx_a, la, ia)

            @pl.when(chunk(i1) < NCHUNKS)
            def _():
                wait_load(i1, buf_b, idx_b, lb, ib)
                scatter(buf_b, idx_b)

            return carry

        lax.fori_loop(0, CPW // 2, body, 0)
        plsc.subcore_barrier()

        @pl.when(sid == 0)
        def _():
            pltpu.sync_copy(acc_sh, out_hbm.at[cid])

    return _sc_gather, _sc_scatter


def _edge_body(ea_ref, xj_ref, w1_ref, b1_ref, t_ref, s_ref, out_ref):
    h = jnp.dot(ea_ref[...], w1_ref[...], preferred_element_type=jnp.float32)
    h = jnp.maximum(h + b1_ref[...], 0.0)
    p = jnp.dot(xj_ref[...], t_ref[...], preferred_element_type=jnp.float32)
    htile = jnp.concatenate([h] * HID, axis=1)
    g = p[:, :PW] * htile
    msg = jnp.dot(g, s_ref[...], preferred_element_type=jnp.float32)
    msg = msg + p[:, PW:]
    onec = (lax.broadcasted_iota(jnp.int32, (BE, HID), 1) == 0)
    pad = jnp.zeros((BE, MSGW - 2 * HID), dtype=jnp.float32)
    out_ref[...] = jnp.concatenate([msg, onec.astype(jnp.float32), pad], axis=1)


def _edge_kernel(edge_attr, xj, w1, b1_2d, t2, s):
    return pl.pallas_call(
        _edge_body,
        grid=(N_EDGES // BE,),
        in_specs=[
            pl.BlockSpec((BE, HID), lambda i: (i, 0)),
            pl.BlockSpec((BE, IN_CH), lambda i: (i, 0)),
            pl.BlockSpec((HID, KH), lambda i: (0, 0)),
            pl.BlockSpec((1, KH), lambda i: (0, 0)),
            pl.BlockSpec((IN_CH, PW + HID), lambda i: (0, 0)),
            pl.BlockSpec((PW, HID), lambda i: (0, 0)),
        ],
        out_specs=pl.BlockSpec((BE, MSGW), lambda i: (i, 0)),
        out_shape=jax.ShapeDtypeStruct((N_EDGES, MSGW), jnp.float32),
    )(edge_attr, xj, w1, b1_2d, t2, s)


def _final_body(p_ref, x_ref, root_ref, bias_ref, wfc_ref, bfc_ref, out_ref):
    s = p_ref[0] + p_ref[1]
    sums = s[:, :HID]
    cnt = s[:, HID:HID + 1]
    aggr = sums / jnp.maximum(cnt, 1.0)
    conv = aggr + jnp.dot(x_ref[...], root_ref[...],
                          preferred_element_type=jnp.float32) + bias_ref[...]
    conv = jnp.maximum(conv, 0.0)
    out_ref[...] = jnp.dot(conv, wfc_ref[...],
                           preferred_element_type=jnp.float32) + bfc_ref[...]


def _final_kernel(partials, x, root, bias_2d, wfc, bfc_2d):
    return pl.pallas_call(
        _final_body,
        grid=(N_NODES // BN,),
        in_specs=[
            pl.BlockSpec((2, BN, MSGW), lambda i: (0, i, 0)),
            pl.BlockSpec((BN, IN_CH), lambda i: (i, 0)),
            pl.BlockSpec((IN_CH, HID), lambda i: (0, 0)),
            pl.BlockSpec((1, HID), lambda i: (0, 0)),
            pl.BlockSpec((HID, 1), lambda i: (0, 0)),
            pl.BlockSpec((1, 1), lambda i: (0, 0)),
        ],
        out_specs=pl.BlockSpec((BN, 1), lambda i: (i, 0)),
        out_shape=jax.ShapeDtypeStruct((N_NODES, 1), jnp.float32),
    )(partials, x, root, bias_2d, wfc, bfc_2d)


def kernel(x, edge_index, edge_attr, W1, b1, W2, b2, root, bias, Wfc, bfc):
    src = edge_index[0].astype(jnp.int32)
    dst = edge_index[1].astype(jnp.int32)

    t = jnp.transpose(W2.reshape(KH, IN_CH, HID), (1, 2, 0)).reshape(IN_CH, PW)
    t2 = jnp.concatenate([t, b2.reshape(IN_CH, HID)], axis=1)
    s = jnp.repeat(jnp.eye(HID, dtype=jnp.float32), KH, axis=0)

    sc_gather, sc_scatter = _sc_kernels()
    xj = sc_gather(x, src)
    msg = _edge_kernel(edge_attr, xj, W1, b1.reshape(1, KH), t2, s)
    zeros = jnp.zeros((N_NODES, MSGW), dtype=jnp.float32)
    partials = sc_scatter(msg, dst, zeros)
    return _final_kernel(partials, x, root, bias.reshape(1, HID),
                         Wfc, bfc.reshape(1, 1))


# speedup vs baseline: 1.4806x; 1.0652x over previous
import functools

import jax
import jax.numpy as jnp
from jax import lax
from jax.experimental import pallas as pl
from jax.experimental.pallas import tpu as pltpu
from jax.experimental.pallas import tpu_sc as plsc

N_NODES = 10000
N_EDGES = 160000
IN_CH = 128
HID = 16
KH = 64
PW = HID * KH
MSGW = 128

CHUNK = 128
NW = 32
NCHUNKS = N_EDGES // CHUNK
CPW = (NCHUNKS + NW - 1) // NW

BE = 1000
BN = 1000


@functools.lru_cache(maxsize=None)
def _sc_kernels():
    mesh = plsc.VectorSubcoreMesh(core_axis_name="c", subcore_axis_name="s")

    @functools.partial(
        pl.kernel,
        out_type=jax.ShapeDtypeStruct((N_EDGES, IN_CH), jnp.float32),
        mesh=mesh,
        scratch_types=[
            pltpu.VMEM((CHUNK,), jnp.int32),
            pltpu.VMEM((CHUNK,), jnp.int32),
            pltpu.VMEM((CHUNK, IN_CH), jnp.float32),
            pltpu.VMEM((CHUNK, IN_CH), jnp.float32),
            pltpu.SemaphoreType.DMA,
            pltpu.SemaphoreType.DMA,
            pltpu.SemaphoreType.DMA,
            pltpu.SemaphoreType.DMA,
            pltpu.SemaphoreType.DMA,
        ],
    )
    def _sc_gather(x_hbm, src_hbm, out_hbm, idx_a, idx_b, rows_a, rows_b,
                   ia, ib, g, oa, ob):
        wid = lax.axis_index("s") * 2 + lax.axis_index("c")

        def chunk(i):
            return wid + i * NW

        def start_idx(i, idx, isem):
            pltpu.async_copy(src_hbm.at[pl.ds(chunk(i) * CHUNK, CHUNK)],
                             idx, isem)

        def wait_idx(i, idx, isem):
            pltpu.make_async_copy(src_hbm.at[pl.ds(chunk(i) * CHUNK, CHUNK)],
                                  idx, isem).wait()

        def gather(idx, rows):
            pltpu.async_copy(x_hbm.at[idx], rows, g)
            pltpu.make_async_copy(x_hbm.at[idx], rows, g).wait()

        def start_out(i, rows, o):
            pltpu.async_copy(rows,
                             out_hbm.at[pl.ds(chunk(i) * CHUNK, CHUNK)], o)

        def wait_out(i, rows, o):
            pltpu.make_async_copy(rows,
                                  out_hbm.at[pl.ds(chunk(i) * CHUNK, CHUNK)],
                                  o).wait()

        @pl.when(chunk(0) < NCHUNKS)
        def _():
            start_idx(0, idx_a, ia)

        def body(it, carry):
            i0 = 2 * it
            i1 = i0 + 1

            @pl.when(chunk(i1) < NCHUNKS)
            def _():
                start_idx(i1, idx_b, ib)

            @pl.when(chunk(i0) < NCHUNKS)
            def _():
                wait_idx(i0, idx_a, ia)

                @pl.when(it >= 1)
                def _():
                    wait_out(i0 - 2, rows_a, oa)

                gather(idx_a, rows_a)
                start_out(i0, rows_a, oa)

            @pl.when(chunk(i1 + 1) < NCHUNKS)
            def _():
                start_idx(i1 + 1, idx_a, ia)

            @pl.when(chunk(i1) < NCHUNKS)
            def _():
                wait_idx(i1, idx_b, ib)

                @pl.when(it >= 1)
                def _():
                    wait_out(i1 - 2, rows_b, ob)

                gather(idx_b, rows_b)
                start_out(i1, rows_b, ob)

            return carry

        lax.fori_loop(0, CPW // 2, body, 0)

        @pl.when(chunk(CPW - 2) < NCHUNKS)
        def _():
            wait_out(CPW - 2, rows_a, oa)

        @pl.when(chunk(CPW - 1) < NCHUNKS)
        def _():
            wait_out(CPW - 1, rows_b, ob)

        @pl.when(chunk(CPW - 1) >= NCHUNKS)
        def _():
            @pl.when(chunk(CPW - 3) < NCHUNKS)
            def _():
                wait_out(CPW - 3, rows_b, ob)

    @functools.partial(
        pl.kernel,
        out_type=jax.ShapeDtypeStruct((2, N_NODES, MSGW), jnp.float32),
        mesh=mesh,
        scratch_types=[
            pltpu.VMEM((CHUNK,), jnp.int32),
            pltpu.VMEM((CHUNK,), jnp.int32),
            pltpu.VMEM((CHUNK, MSGW), jnp.float32),
            pltpu.VMEM((CHUNK, MSGW), jnp.float32),
            pltpu.SemaphoreType.DMA,
            pltpu.SemaphoreType.DMA,
            pltpu.SemaphoreType.DMA,
            pltpu.SemaphoreType.DMA,
            pltpu.VMEM_SHARED((N_NODES, MSGW), jnp.float32),
        ],
    )
    def _sc_scatter(msg_hbm, dst_hbm, zeros_hbm, out_hbm, idx_a, idx_b,
                    buf_a, buf_b, la, lb, ia, ib, acc_sh):
        cid = lax.axis_index("c")
        sid = lax.axis_index("s")
        wid = sid * 2 + cid

        @pl.when(sid == 0)
        def _():
            pltpu.sync_copy(zeros_hbm, acc_sh)

        plsc.subcore_barrier()

        def chunk(i):
            return wid + i * NW

        def start_load(i, buf, idx, l, isem):
            j = chunk(i)
            pltpu.async_copy(msg_hbm.at[pl.ds(j * CHUNK, CHUNK)], buf, l)
            pltpu.async_copy(dst_hbm.at[pl.ds(j * CHUNK, CHUNK)], idx, isem)

        def wait_load(i, buf, idx, l, isem):
            j = chunk(i)
            pltpu.make_async_copy(msg_hbm.at[pl.ds(j * CHUNK, CHUNK)], buf,
                                  l).wait()
            pltpu.make_async_copy(dst_hbm.at[pl.ds(j * CHUNK, CHUNK)], idx,
                                  isem).wait()

        def scatter(buf, idx):
            pltpu.sync_copy(buf, acc_sh.at[idx], add=True)

        @pl.when(chunk(0) < NCHUNKS)
        def _():
            start_load(0, buf_a, idx_a, la, ia)

        def body(it, carry):
            i0 = 2 * it
            i1 = i0 + 1

            @pl.when(chunk(i1) < NCHUNKS)
            def _():
                start_load(i1, buf_b, idx_b, lb, ib)

            @pl.when(chunk(i0) < NCHUNKS)
            def _():
                wait_load(i0, buf_a, idx_a, la, ia)
                scatter(buf_a, idx_a)

            @pl.when(chunk(i1 + 1) < NCHUNKS)
            def _():
                start_load(i1 + 1, buf_a, idx_a, la, ia)

            @pl.when(chunk(i1) < NCHUNKS)
            def _():
                wait_load(i1, buf_b, idx_b, lb, ib)
                scatter(buf_b, idx_b)

            return carry

        lax.fori_loop(0, CPW // 2, body, 0)
        plsc.subcore_barrier()

        @pl.when(sid == 0)
        def _():
            pltpu.sync_copy(acc_sh, out_hbm.at[cid])

    return _sc_gather, _sc_scatter


def _edge_body(ea_ref, xj_ref, w1_ref, b1_ref, t_ref, s_ref, out_ref):
    h = jnp.dot(ea_ref[...], w1_ref[...], preferred_element_type=jnp.float32)
    h = jnp.maximum(h + b1_ref[...], 0.0)
    p = jnp.dot(xj_ref[...], t_ref[...], preferred_element_type=jnp.float32)
    htile = jnp.concatenate([h] * HID, axis=1)
    g = p[:, :PW] * htile
    msg = jnp.dot(g, s_ref[...], preferred_element_type=jnp.float32)
    msg = msg + p[:, PW:]
    onec = (lax.broadcasted_iota(jnp.int32, (BE, HID), 1) == 0)
    pad = jnp.zeros((BE, MSGW - 2 * HID), dtype=jnp.float32)
    out_ref[...] = jnp.concatenate([msg, onec.astype(jnp.float32), pad], axis=1)


def _edge_kernel(edge_attr, xj, w1, b1_2d, t2, s):
    return pl.pallas_call(
        _edge_body,
        grid=(N_EDGES // BE,),
        in_specs=[
            pl.BlockSpec((BE, HID), lambda i: (i, 0)),
            pl.BlockSpec((BE, IN_CH), lambda i: (i, 0)),
            pl.BlockSpec((HID, KH), lambda i: (0, 0)),
            pl.BlockSpec((1, KH), lambda i: (0, 0)),
            pl.BlockSpec((IN_CH, PW + HID), lambda i: (0, 0)),
            pl.BlockSpec((PW, HID), lambda i: (0, 0)),
        ],
        out_specs=pl.BlockSpec((BE, MSGW), lambda i: (i, 0)),
        out_shape=jax.ShapeDtypeStruct((N_EDGES, MSGW), jnp.float32),
    )(edge_attr, xj, w1, b1_2d, t2, s)


def _final_body(p_ref, x_ref, root_ref, bias_ref, wfc_ref, bfc_ref, out_ref):
    s = p_ref[0] + p_ref[1]
    sums = s[:, :HID]
    cnt = s[:, HID:HID + 1]
    aggr = sums / jnp.maximum(cnt, 1.0)
    conv = aggr + jnp.dot(x_ref[...], root_ref[...],
                          preferred_element_type=jnp.float32) + bias_ref[...]
    conv = jnp.maximum(conv, 0.0)
    out_ref[...] = jnp.dot(conv, wfc_ref[...],
                           preferred_element_type=jnp.float32) + bfc_ref[...]


def _final_kernel(partials, x, root, bias_2d, wfc, bfc_2d):
    return pl.pallas_call(
        _final_body,
        grid=(N_NODES // BN,),
        in_specs=[
            pl.BlockSpec((2, BN, MSGW), lambda i: (0, i, 0)),
            pl.BlockSpec((BN, IN_CH), lambda i: (i, 0)),
            pl.BlockSpec((IN_CH, HID), lambda i: (0, 0)),
            pl.BlockSpec((1, HID), lambda i: (0, 0)),
            pl.BlockSpec((HID, 1), lambda i: (0, 0)),
            pl.BlockSpec((1, 1), lambda i: (0, 0)),
        ],
        out_specs=pl.BlockSpec((BN, 1), lambda i: (i, 0)),
        out_shape=jax.ShapeDtypeStruct((N_NODES, 1), jnp.float32),
    )(partials, x, root, bias_2d, wfc, bfc_2d)


def kernel(x, edge_index, edge_attr, W1, b1, W2, b2, root, bias, Wfc, bfc):
    src = edge_index[0].astype(jnp.int32)
    dst = edge_index[1].astype(jnp.int32)

    t = jnp.transpose(W2.reshape(KH, IN_CH, HID), (1, 2, 0)).reshape(IN_CH, PW)
    t2 = jnp.concatenate([t, b2.reshape(IN_CH, HID)], axis=1)
    s = jnp.repeat(jnp.eye(HID, dtype=jnp.float32), KH, axis=0)

    sc_gather, sc_scatter = _sc_kernels()
    xj = sc_gather(x, src)
    msg = _edge_kernel(edge_attr, xj, W1, b1.reshape(1, KH), t2, s)
    zeros = jnp.zeros((N_NODES, MSGW), dtype=jnp.float32)
    partials = sc_scatter(msg, dst, zeros)
    return _final_kernel(partials, x, root, bias.reshape(1, HID),
                         Wfc, bfc.reshape(1, 1))


# speedup vs baseline: 1.4958x; 1.0103x over previous
import functools

import jax
import jax.numpy as jnp
from jax import lax
from jax.experimental import pallas as pl
from jax.experimental.pallas import tpu as pltpu
from jax.experimental.pallas import tpu_sc as plsc

N_NODES = 10000
N_EDGES = 160000
IN_CH = 128
HID = 16
KH = 64
PW = HID * KH
MSGW = 128

CHUNK = 128
NW = 32
NCHUNKS = N_EDGES // CHUNK
CPW = (NCHUNKS + NW - 1) // NW

BE = 1000
BN = 1000


@functools.lru_cache(maxsize=None)
def _sc_kernels(n_edges=N_EDGES):
    mesh = plsc.VectorSubcoreMesh(core_axis_name="c", subcore_axis_name="s")
    NCHUNKS = n_edges // CHUNK
    CPW = -(-NCHUNKS // NW) * 2 // 2
    CPW = CPW + (CPW % 2)

    @functools.partial(
        pl.kernel,
        out_type=jax.ShapeDtypeStruct((n_edges, IN_CH), jnp.float32),
        mesh=mesh,
        scratch_types=[
            pltpu.VMEM((CHUNK,), jnp.int32),
            pltpu.VMEM((CHUNK,), jnp.int32),
            pltpu.VMEM((CHUNK, IN_CH), jnp.float32),
            pltpu.VMEM((CHUNK, IN_CH), jnp.float32),
            pltpu.SemaphoreType.DMA,
            pltpu.SemaphoreType.DMA,
            pltpu.SemaphoreType.DMA,
            pltpu.SemaphoreType.DMA,
            pltpu.SemaphoreType.DMA,
        ],
    )
    def _sc_gather(x_hbm, src_hbm, out_hbm, idx_a, idx_b, rows_a, rows_b,
                   ia, ib, g, oa, ob):
        wid = lax.axis_index("s") * 2 + lax.axis_index("c")

        def chunk(i):
            return wid + i * NW

        def start_idx(i, idx, isem):
            pltpu.async_copy(src_hbm.at[pl.ds(chunk(i) * CHUNK, CHUNK)],
                             idx, isem)

        def wait_idx(i, idx, isem):
            pltpu.make_async_copy(src_hbm.at[pl.ds(chunk(i) * CHUNK, CHUNK)],
                                  idx, isem).wait()

        def gather(idx, rows):
            pltpu.async_copy(x_hbm.at[idx], rows, g)
            pltpu.make_async_copy(x_hbm.at[idx], rows, g).wait()

        def start_out(i, rows, o):
            pltpu.async_copy(rows,
                             out_hbm.at[pl.ds(chunk(i) * CHUNK, CHUNK)], o)

        def wait_out(i, rows, o):
            pltpu.make_async_copy(rows,
                                  out_hbm.at[pl.ds(chunk(i) * CHUNK, CHUNK)],
                                  o).wait()

        @pl.when(chunk(0) < NCHUNKS)
        def _():
            start_idx(0, idx_a, ia)

        def body(it, carry):
            i0 = 2 * it
            i1 = i0 + 1

            @pl.when(chunk(i1) < NCHUNKS)
            def _():
                start_idx(i1, idx_b, ib)

            @pl.when(chunk(i0) < NCHUNKS)
            def _():
                wait_idx(i0, idx_a, ia)

                @pl.when(it >= 1)
                def _():
                    wait_out(i0 - 2, rows_a, oa)

                gather(idx_a, rows_a)
                start_out(i0, rows_a, oa)

            @pl.when(chunk(i1 + 1) < NCHUNKS)
            def _():
                start_idx(i1 + 1, idx_a, ia)

            @pl.when(chunk(i1) < NCHUNKS)
            def _():
                wait_idx(i1, idx_b, ib)

                @pl.when(it >= 1)
                def _():
                    wait_out(i1 - 2, rows_b, ob)

                gather(idx_b, rows_b)
                start_out(i1, rows_b, ob)

            return carry

        lax.fori_loop(0, CPW // 2, body, 0)

        @pl.when(chunk(CPW - 2) < NCHUNKS)
        def _():
            wait_out(CPW - 2, rows_a, oa)

        @pl.when(chunk(CPW - 1) < NCHUNKS)
        def _():
            wait_out(CPW - 1, rows_b, ob)

        @pl.when(chunk(CPW - 1) >= NCHUNKS)
        def _():
            @pl.when(chunk(CPW - 3) < NCHUNKS)
            def _():
                wait_out(CPW - 3, rows_b, ob)

    @functools.partial(
        pl.kernel,
        out_type=jax.ShapeDtypeStruct((2, N_NODES, MSGW), jnp.float32),
        mesh=mesh,
        scratch_types=[
            pltpu.VMEM((CHUNK,), jnp.int32),
            pltpu.VMEM((CHUNK,), jnp.int32),
            pltpu.VMEM((CHUNK, MSGW), jnp.float32),
            pltpu.VMEM((CHUNK, MSGW), jnp.float32),
            pltpu.SemaphoreType.DMA,
            pltpu.SemaphoreType.DMA,
            pltpu.SemaphoreType.DMA,
            pltpu.SemaphoreType.DMA,
            pltpu.VMEM_SHARED((N_NODES, MSGW), jnp.float32),
        ],
    )
    def _sc_scatter(msg_hbm, dst_hbm, zeros_hbm, out_hbm, idx_a, idx_b,
                    buf_a, buf_b, la, lb, ia, ib, acc_sh):
        cid = lax.axis_index("c")
        sid = lax.axis_index("s")
        wid = sid * 2 + cid

        @pl.when(sid == 0)
        def _():
            pltpu.sync_copy(zeros_hbm, acc_sh)

        plsc.subcore_barrier()

        def chunk(i):
            return wid + i * NW

        def start_load(i, buf, idx, l, isem):
            j = chunk(i)
            pltpu.async_copy(msg_hbm.at[pl.ds(j * CHUNK, CHUNK)], buf, l)
            pltpu.async_copy(dst_hbm.at[pl.ds(j * CHUNK, CHUNK)], idx, isem)

        def wait_load(i, buf, idx, l, isem):
            j = chunk(i)
            pltpu.make_async_copy(msg_hbm.at[pl.ds(j * CHUNK, CHUNK)], buf,
                                  l).wait()
            pltpu.make_async_copy(dst_hbm.at[pl.ds(j * CHUNK, CHUNK)], idx,
                                  isem).wait()

        def scatter(buf, idx):
            pltpu.sync_copy(buf, acc_sh.at[idx], add=True)

        @pl.when(chunk(0) < NCHUNKS)
        def _():
            start_load(0, buf_a, idx_a, la, ia)

        def body(it, carry):
            i0 = 2 * it
            i1 = i0 + 1

            @pl.when(chunk(i1) < NCHUNKS)
            def _():
                start_load(i1, buf_b, idx_b, lb, ib)

            @pl.when(chunk(i0) < NCHUNKS)
            def _():
                wait_load(i0, buf_a, idx_a, la, ia)
                scatter(buf_a, idx_a)

            @pl.when(chunk(i1 + 1) < NCHUNKS)
            def _():
                start_load(i1 + 1, buf_a, idx_a, la, ia)

            @pl.when(chunk(i1) < NCHUNKS)
            def _():
                wait_load(i1, buf_b, idx_b, lb, ib)
                scatter(buf_b, idx_b)

            return carry

        lax.fori_loop(0, CPW // 2, body, 0)
        plsc.subcore_barrier()

        @pl.when(sid == 0)
        def _():
            pltpu.sync_copy(acc_sh, out_hbm.at[cid])

    return _sc_gather, _sc_scatter


def _edge_body(ea_ref, xj_ref, w1_ref, b1_ref, t_ref, s_ref, out_ref):
    h = jnp.dot(ea_ref[...], w1_ref[...], preferred_element_type=jnp.float32)
    h = jnp.maximum(h + b1_ref[...], 0.0)
    p = jnp.dot(xj_ref[...], t_ref[...], preferred_element_type=jnp.float32)
    htile = jnp.concatenate([h] * HID, axis=1)
    g = p[:, :PW] * htile
    msg = jnp.dot(g, s_ref[...], preferred_element_type=jnp.float32)
    msg = msg + p[:, PW:]
    onec = (lax.broadcasted_iota(jnp.int32, (BE, HID), 1) == 0)
    pad = jnp.zeros((BE, MSGW - 2 * HID), dtype=jnp.float32)
    out_ref[...] = jnp.concatenate([msg, onec.astype(jnp.float32), pad], axis=1)


def _edge_kernel(edge_attr, xj, w1, b1_2d, t2, s):
    n_edges = xj.shape[0]
    return pl.pallas_call(
        _edge_body,
        grid=(n_edges // BE,),
        in_specs=[
            pl.BlockSpec((BE, HID), lambda i: (i, 0)),
            pl.BlockSpec((BE, IN_CH), lambda i: (i, 0)),
            pl.BlockSpec((HID, KH), lambda i: (0, 0)),
            pl.BlockSpec((1, KH), lambda i: (0, 0)),
            pl.BlockSpec((IN_CH, PW + HID), lambda i: (0, 0)),
            pl.BlockSpec((PW, HID), lambda i: (0, 0)),
        ],
        out_specs=pl.BlockSpec((BE, MSGW), lambda i: (i, 0)),
        out_shape=jax.ShapeDtypeStruct((n_edges, MSGW), jnp.float32),
    )(edge_attr, xj, w1, b1_2d, t2, s)


def _final_body(p_ref, q_ref, x_ref, root_ref, bias_ref, wfc_ref, bfc_ref,
                out_ref):
    s = p_ref[0] + p_ref[1] + q_ref[0] + q_ref[1]
    sums = s[:, :HID]
    cnt = s[:, HID:HID + 1]
    aggr = sums / jnp.maximum(cnt, 1.0)
    conv = aggr + jnp.dot(x_ref[...], root_ref[...],
                          preferred_element_type=jnp.float32) + bias_ref[...]
    conv = jnp.maximum(conv, 0.0)
    out_ref[...] = jnp.dot(conv, wfc_ref[...],
                           preferred_element_type=jnp.float32) + bfc_ref[...]


def _final_kernel(partials0, partials1, x, root, bias_2d, wfc, bfc_2d):
    return pl.pallas_call(
        _final_body,
        grid=(N_NODES // BN,),
        in_specs=[
            pl.BlockSpec((2, BN, MSGW), lambda i: (0, i, 0)),
            pl.BlockSpec((2, BN, MSGW), lambda i: (0, i, 0)),
            pl.BlockSpec((BN, IN_CH), lambda i: (i, 0)),
            pl.BlockSpec((IN_CH, HID), lambda i: (0, 0)),
            pl.BlockSpec((1, HID), lambda i: (0, 0)),
            pl.BlockSpec((HID, 1), lambda i: (0, 0)),
            pl.BlockSpec((1, 1), lambda i: (0, 0)),
        ],
        out_specs=pl.BlockSpec((BN, 1), lambda i: (i, 0)),
        out_shape=jax.ShapeDtypeStruct((N_NODES, 1), jnp.float32),
    )(partials0, partials1, x, root, bias_2d, wfc, bfc_2d)


def kernel(x, edge_index, edge_attr, W1, b1, W2, b2, root, bias, Wfc, bfc):
    src = edge_index[0].astype(jnp.int32)
    dst = edge_index[1].astype(jnp.int32)

    t = jnp.transpose(W2.reshape(KH, IN_CH, HID), (1, 2, 0)).reshape(IN_CH, PW)
    t2 = jnp.concatenate([t, b2.reshape(IN_CH, HID)], axis=1)
    s = jnp.repeat(jnp.eye(HID, dtype=jnp.float32), KH, axis=0)

    sc_gather, sc_scatter = _sc_kernels(N_EDGES // 2)
    zeros = jnp.zeros((N_NODES, MSGW), dtype=jnp.float32)
    b1_2d = b1.reshape(1, KH)
    half = N_EDGES // 2
    parts = []
    for lo in (0, half):
        xj = sc_gather(x, lax.dynamic_slice_in_dim(src, lo, half))
        msg = _edge_kernel(
            lax.dynamic_slice_in_dim(edge_attr, lo, half), xj, W1, b1_2d,
            t2, s)
        parts.append(sc_scatter(msg, lax.dynamic_slice_in_dim(dst, lo, half),
                                zeros))
    return _final_kernel(parts[0], parts[1], x, root, bias.reshape(1, HID),
                         Wfc, bfc.reshape(1, 1))
